# baseline (speedup 1.0000x reference)
import functools

import jax
import jax.numpy as jnp
from jax import lax
from jax.experimental import pallas as pl
from jax.experimental.pallas import tpu as pltpu
from jax.experimental.pallas import tpu_sc as plsc

H = 128
LANES = 16
NH = H // LANES


def _allsum(v):
  idx = lax.iota(jnp.int32, LANES)
  for sh in (8, 4, 2, 1):
    v = v + v.at[idx ^ sh].get(mode="promise_in_bounds")
  return v


def _rsqrt(x):
  i = plsc.bitcast(x, jnp.int32)
  i = jnp.int32(0x5F3759DF) - (i >> 1)
  y = plsc.bitcast(i, jnp.float32)
  for _ in range(3):
    y = y * (1.5 - 0.5 * x * y * y)
  return y


def _build_kernel(N, NW, T):
  per_w = N // NW
  n_blocks = per_w // T
  mesh = plsc.VectorSubcoreMesh(core_axis_name="c", subcore_axis_name="s")

  @functools.partial(
      pl.kernel,
      out_type=jax.ShapeDtypeStruct((N, H), jnp.float32),
      mesh=mesh,
      compiler_params=pltpu.CompilerParams(needs_layout_passes=False),
      scratch_types=[
          pltpu.VMEM((T,), jnp.int32),
          pltpu.VMEM((T,), jnp.int32),
          pltpu.VMEM((T,), jnp.int32),
          pltpu.VMEM((T,), jnp.int32),
          pltpu.VMEM((T, H), jnp.float32),
          pltpu.VMEM((T, H), jnp.float32),
          pltpu.VMEM((T, H), jnp.float32),
          pltpu.VMEM((T, H), jnp.float32),
          pltpu.VMEM((H,), jnp.float32),
          pltpu.VMEM((H,), jnp.float32),
          pltpu.SemaphoreType.DMA,
          pltpu.SemaphoreType.DMA,
          pltpu.SemaphoreType.DMA,
          pltpu.SemaphoreType.DMA,
      ],
  )
  def k(wid_h, sid_h, aid_h, pid_h, wt_h, st_h, at_h, pt_h, g_h, b_h, out_h,
        widx, sidx, aidx, pidx, wbuf, sbuf, abuf, pbuf, gv, bv,
        sem0, sem1, sem2, sem3):
    w = lax.axis_index("s") * 2 + lax.axis_index("c")
    base = w * per_w
    pltpu.sync_copy(g_h, gv)
    pltpu.sync_copy(b_h, bv)

    def block(j, carry):
      b0 = base + j * T
      pltpu.sync_copy(wid_h.at[pl.ds(b0, T)], widx)
      pltpu.sync_copy(sid_h.at[pl.ds(b0, T)], sidx)
      pltpu.sync_copy(aid_h.at[pl.ds(b0, T)], aidx)
      pltpu.sync_copy(pid_h.at[pl.ds(b0, T)], pidx)
      cw = pltpu.async_copy(wt_h.at[widx], wbuf, sem0)
      cs = pltpu.async_copy(st_h.at[sidx], sbuf, sem1)
      ca = pltpu.async_copy(at_h.at[aidx], abuf, sem2)
      cp = pltpu.async_copy(pt_h.at[pidx], pbuf, sem3)
      cw.wait()
      cs.wait()
      ca.wait()
      cp.wait()

      def token(t, c):
        vs = []
        tot = None
        ssq = None
        for kk in range(NH):
          sl = pl.ds(kk * LANES, LANES)
          v = wbuf[t, sl] + sbuf[t, sl] + abuf[t, sl] + pbuf[t, sl]
          vs.append(v)
          tot = v if tot is None else tot + v
          ssq = v * v if ssq is None else ssq + v * v
        mv = _allsum(tot) * (1.0 / H)
        ex2 = _allsum(ssq) * (1.0 / H)
        var = ex2 - mv * mv
        inv = _rsqrt(var + 1e-12)
        for kk in range(NH):
          sl = pl.ds(kk * LANES, LANES)
          wbuf[t, sl] = (vs[kk] - mv) * inv * gv[sl] + bv[sl]
        return c

      lax.fori_loop(0, T, token, 0, unroll=False)
      pltpu.sync_copy(wbuf, out_h.at[pl.ds(b0, T)])
      return carry

    lax.fori_loop(0, n_blocks, block, 0, unroll=False)

  return k


def kernel(word_ids, age_ids, seg_ids, posi_ids, word_table, seg_table,
           age_table, posi_table, gamma, beta):
  B, L = word_ids.shape
  N = B * L
  NW = 32
  T = 160
  wid = word_ids.reshape(N).astype(jnp.int32)
  sid = seg_ids.reshape(N).astype(jnp.int32)
  aid = age_ids.reshape(N).astype(jnp.int32)
  pid = posi_ids.reshape(N).astype(jnp.int32)
  k = _build_kernel(N, NW, T)
  out = k(wid, sid, aid, pid,
          word_table.astype(jnp.float32), seg_table.astype(jnp.float32),
          age_table.astype(jnp.float32), posi_table.astype(jnp.float32),
          gamma.astype(jnp.float32), beta.astype(jnp.float32))
  return out.reshape(B, L, H)


# speedup vs baseline: 8.6775x; 8.6775x over previous
import functools

import jax
import jax.numpy as jnp
from jax import lax
from jax.experimental import pallas as pl
from jax.experimental.pallas import tpu as pltpu
from jax.experimental.pallas import tpu_sc as plsc

H = 128
NW = 32
GT = 400
TT = 512
SEG_OFF = 0
AGE_OFF = 2
POS_OFF = 146
KPAD = 768


def _build_sc_gather(N):
  per_w = N // NW
  nb = per_w // GT
  mesh = plsc.VectorSubcoreMesh(core_axis_name="c", subcore_axis_name="s")

  @functools.partial(
      pl.kernel,
      out_type=jax.ShapeDtypeStruct((N, H), jnp.float32),
      mesh=mesh,
      compiler_params=pltpu.CompilerParams(needs_layout_passes=False),
      scratch_types=[
          pltpu.VMEM((per_w,), jnp.int32),
          pltpu.VMEM((GT, H), jnp.float32),
          pltpu.VMEM((GT, H), jnp.float32),
          pltpu.SemaphoreType.DMA,
          pltpu.SemaphoreType.DMA,
          pltpu.SemaphoreType.DMA,
          pltpu.SemaphoreType.DMA,
      ],
  )
  def k(ids_h, tab_h, out_h, idx_v, buf0, buf1, gs0, gs1, ws0, ws1):
    w = lax.axis_index("s") * 2 + lax.axis_index("c")
    base = w * per_w
    pltpu.sync_copy(ids_h.at[pl.ds(base, per_w)], idx_v)
    bufs = (buf0, buf1)
    gsems = (gs0, gs1)
    wsems = (ws0, ws1)
    gd = {}
    wd = {}
    for j in range(nb):
      p = j % 2
      if j >= 2:
        wd[j - 2].wait()
      gd[j] = pltpu.async_copy(
          tab_h.at[idx_v.at[pl.ds(j * GT, GT)]], bufs[p], gsems[p])
      if j >= 1:
        q = (j - 1) % 2
        gd[j - 1].wait()
        wd[j - 1] = pltpu.async_copy(
            bufs[q], out_h.at[pl.ds(base + (j - 1) * GT, GT)], wsems[q])
    q = (nb - 1) % 2
    gd[nb - 1].wait()
    wd[nb - 1] = pltpu.async_copy(
        bufs[q], out_h.at[pl.ds(base + (nb - 1) * GT, GT)], wsems[q])
    wd[nb - 2].wait()
    wd[nb - 1].wait()

  return k


def _tc_body(wrows_ref, sid_ref, aid_ref, pid_ref, tab_ref, g_ref, b_ref,
             o_ref):
  x = wrows_ref[...]
  col = lax.broadcasted_iota(jnp.int32, (TT, KPAD), 1)
  sid = sid_ref[0, 0, :][:, None]
  aid = aid_ref[0, 0, :][:, None]
  pid = pid_ref[0, 0, :][:, None]
  oh = (col == sid + SEG_OFF) | (col == aid + AGE_OFF) | (col == pid + POS_OFF)
  small = lax.dot_general(
      oh.astype(jnp.bfloat16), tab_ref[...],
      (((1,), (0,)), ((), ())), preferred_element_type=jnp.float32)
  x = x + small
  u = jnp.mean(x, axis=1, keepdims=True)
  d = x - u
  var = jnp.mean(d * d, axis=1, keepdims=True)
  y = d * lax.rsqrt(var + 1e-12)
  o_ref[...] = y * g_ref[0, :] + b_ref[0, :]


def _tc_ln(wrows, sid, aid, pid, tab, gamma, beta):
  N = wrows.shape[0]
  nblk = N // TT
  grid = (nblk,)
  return pl.pallas_call(
      _tc_body,
      grid=grid,
      in_specs=[
          pl.BlockSpec((TT, H), lambda j: (j, 0)),
          pl.BlockSpec((1, 1, TT), lambda j: (j, 0, 0)),
          pl.BlockSpec((1, 1, TT), lambda j: (j, 0, 0)),
          pl.BlockSpec((1, 1, TT), lambda j: (j, 0, 0)),
          pl.BlockSpec((KPAD, H), lambda j: (0, 0)),
          pl.BlockSpec((1, H), lambda j: (0, 0)),
          pl.BlockSpec((1, H), lambda j: (0, 0)),
      ],
      out_specs=pl.BlockSpec((TT, H), lambda j: (j, 0)),
      out_shape=jax.ShapeDtypeStruct((N, H), jnp.float32),
      compiler_params=pltpu.CompilerParams(
          dimension_semantics=("arbitrary",)),
  )(wrows, sid.reshape(nblk, 1, TT), aid.reshape(nblk, 1, TT),
    pid.reshape(nblk, 1, TT), tab, gamma.reshape(1, H), beta.reshape(1, H))


def kernel(word_ids, age_ids, seg_ids, posi_ids, word_table, seg_table,
           age_table, posi_table, gamma, beta):
  B, L = word_ids.shape
  N = B * L
  wid = word_ids.reshape(N).astype(jnp.int32)
  sid = seg_ids.reshape(N).astype(jnp.int32)
  aid = age_ids.reshape(N).astype(jnp.int32)
  pid = posi_ids.reshape(N).astype(jnp.int32)
  tab = jnp.zeros((KPAD, H), jnp.bfloat16)
  tab = tab.at[SEG_OFF:SEG_OFF + 2].set(seg_table.astype(jnp.bfloat16))
  tab = tab.at[AGE_OFF:AGE_OFF + 144].set(age_table.astype(jnp.bfloat16))
  tab = tab.at[POS_OFF:POS_OFF + 512].set(posi_table.astype(jnp.bfloat16))

  sc_gather = _build_sc_gather(N)
  wrows = sc_gather(wid, word_table.astype(jnp.float32))
  out = _tc_ln(wrows, sid, aid, pid, tab,
               gamma.astype(jnp.float32), beta.astype(jnp.float32))
  return out.reshape(B, L, H)


# speedup vs baseline: 9.7885x; 1.1280x over previous
import functools

import jax
import jax.numpy as jnp
import numpy as np
from jax import lax
from jax.experimental import pallas as pl
from jax.experimental.pallas import tpu as pltpu
from jax.experimental.pallas import tpu_sc as plsc

H = 128
NW = 32
GT = 400
TT = 512
SEG_OFF = 0
AGE_OFF = 2
KPAD = 256


def _build_sc_gather(N):
  per_w = N // NW
  nb = per_w // GT
  mesh = plsc.VectorSubcoreMesh(core_axis_name="c", subcore_axis_name="s")

  @functools.partial(
      pl.kernel,
      out_type=jax.ShapeDtypeStruct((N, H), jnp.float32),
      mesh=mesh,
      compiler_params=pltpu.CompilerParams(needs_layout_passes=False),
      scratch_types=[
          pltpu.VMEM((per_w,), jnp.int32),
          pltpu.VMEM((GT, H), jnp.float32),
          pltpu.VMEM((GT, H), jnp.float32),
          pltpu.SemaphoreType.DMA,
          pltpu.SemaphoreType.DMA,
          pltpu.SemaphoreType.DMA,
          pltpu.SemaphoreType.DMA,
      ],
  )
  def k(ids_h, tab_h, out_h, idx_v, buf0, buf1, gs0, gs1, ws0, ws1):
    w = lax.axis_index("s") * 2 + lax.axis_index("c")
    base = w * per_w
    pltpu.sync_copy(ids_h.at[pl.ds(base, per_w)], idx_v)
    bufs = (buf0, buf1)
    gsems = (gs0, gs1)
    wsems = (ws0, ws1)
    gd = {}
    wd = {}
    for j in range(nb):
      p = j % 2
      if j >= 2:
        wd[j - 2].wait()
      gd[j] = pltpu.async_copy(
          tab_h.at[idx_v.at[pl.ds(j * GT, GT)]], bufs[p], gsems[p])
      if j >= 1:
        q = (j - 1) % 2
        gd[j - 1].wait()
        wd[j - 1] = pltpu.async_copy(
            bufs[q], out_h.at[pl.ds(base + (j - 1) * GT, GT)], wsems[q])
    q = (nb - 1) % 2
    gd[nb - 1].wait()
    wd[nb - 1] = pltpu.async_copy(
        bufs[q], out_h.at[pl.ds(base + (nb - 1) * GT, GT)], wsems[q])
    wd[nb - 2].wait()
    wd[nb - 1].wait()

  return k


def _tc_body(wrows_ref, sid_ref, aid_ref, pid_ref, tab_ref, ptab_ref, g_ref,
             b_ref, o_ref):
  x = wrows_ref[...]
  col = lax.broadcasted_iota(jnp.int32, (TT, KPAD), 1)
  sid = sid_ref[0, 0, :][:, None]
  aid = aid_ref[0, 0, :][:, None]
  pid = pid_ref[0, 0, :][:, None]
  oh = (col == sid + SEG_OFF) | (col == aid + AGE_OFF)
  small = lax.dot_general(
      oh.astype(jnp.bfloat16), tab_ref[...],
      (((1,), (0,)), ((), ())), preferred_element_type=jnp.float32)
  colp = lax.broadcasted_iota(jnp.int32, (TT, 512), 1)
  ohp = (colp == pid).astype(jnp.bfloat16)
  posi = lax.dot_general(
      ohp, ptab_ref[...],
      (((1,), (0,)), ((), ())), preferred_element_type=jnp.float32)
  x = x + small + posi
  u = jnp.mean(x, axis=1, keepdims=True)
  d = x - u
  var = jnp.mean(d * d, axis=1, keepdims=True)
  y = d * lax.rsqrt(var + 1e-12)
  o_ref[...] = y * g_ref[0, :] + b_ref[0, :]


def _tc_ln(wrows, sid, aid, pid, tab, ptab, gamma, beta):
  N = wrows.shape[0]
  nblk = N // TT
  grid = (nblk,)
  rep = pl.BlockSpec((1, H), lambda j: (0, 0))
  ids = pl.BlockSpec((1, 1, TT), lambda j: (j, 0, 0))
  return pl.pallas_call(
      _tc_body,
      grid=grid,
      in_specs=[
          pl.BlockSpec((TT, H), lambda j: (j, 0)),
          ids, ids, ids,
          pl.BlockSpec((KPAD, H), lambda j: (0, 0)),
          pl.BlockSpec((512, H), lambda j: (0, 0)),
          rep, rep,
      ],
      out_specs=pl.BlockSpec((TT, H), lambda j: (j, 0)),
      out_shape=jax.ShapeDtypeStruct((N, H), jnp.float32),
      compiler_params=pltpu.CompilerParams(
          dimension_semantics=("arbitrary",)),
  )(wrows, sid.reshape(nblk, 1, TT), aid.reshape(nblk, 1, TT),
    pid.reshape(nblk, 1, TT), tab, ptab,
    gamma.reshape(1, H), beta.reshape(1, H))


def kernel(word_ids, age_ids, seg_ids, posi_ids, word_table, seg_table,
           age_table, posi_table, gamma, beta):
  B, L = word_ids.shape
  N = B * L
  wid = word_ids.reshape(N).astype(jnp.int32)
  sid = seg_ids.reshape(N).astype(jnp.int32)
  aid = age_ids.reshape(N).astype(jnp.int32)
  pid = posi_ids.reshape(N).astype(jnp.int32)
  tab = jnp.zeros((KPAD, H), jnp.bfloat16)
  tab = tab.at[SEG_OFF:SEG_OFF + 2].set(seg_table.astype(jnp.bfloat16))
  tab = tab.at[AGE_OFF:AGE_OFF + 144].set(age_table.astype(jnp.bfloat16))
  ptab = posi_table.astype(jnp.bfloat16)

  sc_gather = _build_sc_gather(N)
  wrows = sc_gather(wid, word_table.astype(jnp.float32))
  out = _tc_ln(wrows, sid, aid, pid, tab, ptab,
               gamma.astype(jnp.float32), beta.astype(jnp.float32))
  return out.reshape(B, L, H)
